# initial kernel scaffold (unmeasured)
import jax
import jax.numpy as jnp
from jax import lax
from jax.experimental import pallas as pl
from jax.experimental.pallas import tpu as pltpu

T = 1024
D = 2048
V_LOC = 16384
V_GLOB = 2 * V_LOC
N_CHUNK = 8
CHUNK = V_LOC // N_CHUNK


def kernel(x, W):
    def body(x_ref, w_hbm, out_hbm,
             w_buf, l_buf, n_buf, m_ref, s_ref, ms_send, ms_recv,
             w_sems, st_sems, send_sems, recv_sems,
             ms_send_sem, ms_recv_sem, rd_sems, wr_sems):
        my_x = lax.axis_index("x")
        my_y = lax.axis_index("y")
        my_z = lax.axis_index("z")
        peer = (1 - my_x, my_y, my_z)

        barrier_sem = pltpu.get_barrier_semaphore()
        pl.semaphore_signal(barrier_sem, inc=1, device_id=peer,
                            device_id_type=pl.DeviceIdType.MESH)
        pl.semaphore_wait(barrier_sem, 1)

        my_base = my_x * V_LOC
        peer_base = (1 - my_x) * V_LOC

        m_ref[...] = jnp.full((T, 1), -1e30, jnp.float32)
        s_ref[...] = jnp.zeros((T, 1), jnp.float32)

        def w_copy(k):
            return pltpu.make_async_copy(
                w_hbm.at[:, pl.ds(k * CHUNK, CHUNK)],
                w_buf.at[k % 2],
                w_sems.at[k % 2],
            )

        rdmas = [None] * N_CHUNK
        stores = [None] * N_CHUNK

        w_copy(0).start()
        for k in range(N_CHUNK):
            slot = k % 2
            if k + 1 < N_CHUNK:
                w_copy(k + 1).start()
            w_copy(k).wait()

            if k >= 2:
                rdmas[k - 2].wait_send()
                stores[k - 2].wait()

            logits = jnp.dot(x_ref[...], w_buf[slot],
                             preferred_element_type=jnp.float32)
            m_old = m_ref[...]
            m_new = jnp.maximum(m_old, jnp.max(logits, axis=1, keepdims=True))
            e_sum = jnp.sum(jnp.exp(logits - m_new), axis=1, keepdims=True)
            s_ref[...] = s_ref[...] * jnp.exp(m_old - m_new) + e_sum
            m_ref[...] = m_new
            l_buf[slot] = logits

            stores[k] = pltpu.make_async_copy(
                l_buf.at[slot],
                out_hbm.at[:, pl.ds(my_base + k * CHUNK, CHUNK)],
                st_sems.at[slot],
            )
            stores[k].start()
            rdmas[k] = pltpu.make_async_remote_copy(
                src_ref=l_buf.at[slot],
                dst_ref=out_hbm.at[:, pl.ds(my_base + k * CHUNK, CHUNK)],
                send_sem=send_sems.at[k],
                recv_sem=recv_sems.at[k],
                device_id=peer,
                device_id_type=pl.DeviceIdType.MESH,
            )
            rdmas[k].start()

        for k in (N_CHUNK - 2, N_CHUNK - 1):
            rdmas[k].wait_send()
            stores[k].wait()

        ms_send[:, 0:128] = jnp.broadcast_to(m_ref[...], (T, 128))
        ms_send[:, 128:256] = jnp.broadcast_to(s_ref[...], (T, 128))
        ms_rdma = pltpu.make_async_remote_copy(
            src_ref=ms_send,
            dst_ref=ms_recv,
            send_sem=ms_send_sem,
            recv_sem=ms_recv_sem,
            device_id=peer,
            device_id_type=pl.DeviceIdType.MESH,
        )
        ms_rdma.start()
        ms_rdma.wait()

        m_loc = m_ref[...]
        s_loc = s_ref[...]
        m_rem = ms_recv[:, 0:1]
        s_rem = ms_recv[:, 128:129]
        m_g = jnp.maximum(m_loc, m_rem)
        s_g = s_loc * jnp.exp(m_loc - m_g) + s_rem * jnp.exp(m_rem - m_g)
        inv_s = 1.0 / s_g

        order = [(my_base + k * CHUNK, None) for k in range(N_CHUNK)]
        order += [(peer_base + k * CHUNK, k) for k in range(N_CHUNK)]

        writes = [None] * len(order)
        for j, (col, pk) in enumerate(order):
            slot = j % 2
            if pk is not None:
                rdmas[pk].wait_recv()
            if j >= 2:
                writes[j - 2].wait()
            rd = pltpu.make_async_copy(
                out_hbm.at[:, pl.ds(col, CHUNK)],
                n_buf.at[slot],
                rd_sems.at[slot],
            )
            rd.start()
            rd.wait()
            n_buf[slot] = jnp.exp(n_buf[slot] - m_g) * inv_s
            writes[j] = pltpu.make_async_copy(
                n_buf.at[slot],
                out_hbm.at[:, pl.ds(col, CHUNK)],
                wr_sems.at[slot],
            )
            writes[j].start()
        writes[-2].wait()
        writes[-1].wait()

    return pl.pallas_call(
        body,
        out_shape=jax.ShapeDtypeStruct((T, V_GLOB), jnp.float32),
        in_specs=[
            pl.BlockSpec(memory_space=pltpu.VMEM),
            pl.BlockSpec(memory_space=pltpu.ANY),
        ],
        out_specs=pl.BlockSpec(memory_space=pltpu.ANY),
        scratch_shapes=[
            pltpu.VMEM((2, D, CHUNK), jnp.float32),
            pltpu.VMEM((2, T, CHUNK), jnp.float32),
            pltpu.VMEM((2, T, CHUNK), jnp.float32),
            pltpu.VMEM((T, 1), jnp.float32),
            pltpu.VMEM((T, 1), jnp.float32),
            pltpu.VMEM((T, 256), jnp.float32),
            pltpu.VMEM((T, 256), jnp.float32),
            pltpu.SemaphoreType.DMA((2,)),
            pltpu.SemaphoreType.DMA((2,)),
            pltpu.SemaphoreType.DMA((N_CHUNK,)),
            pltpu.SemaphoreType.DMA((N_CHUNK,)),
            pltpu.SemaphoreType.DMA,
            pltpu.SemaphoreType.DMA,
            pltpu.SemaphoreType.DMA((2,)),
            pltpu.SemaphoreType.DMA((2,)),
        ],
        compiler_params=pltpu.CompilerParams(collective_id=0),
    )(x, W)


# baseline (device time: 978947 ns/iter reference)
import jax
import jax.numpy as jnp
from jax import lax
from jax.experimental import pallas as pl
from jax.experimental.pallas import tpu as pltpu

T = 1024
D = 2048
V_LOC = 16384
V_GLOB = 2 * V_LOC
N_CHUNK = 16
CHUNK = V_LOC // N_CHUNK


def kernel(x, W):
    def body(x_ref, w_hbm, out_hbm,
             w_buf, l_buf, m_ref, s_ref, ms_send, ms_recv,
             w_sems, st_sems, send_sems, recv_sems,
             ms_send_sem, ms_recv_sem, rd_sems, wr_sems):
        my_x = lax.axis_index("x")
        my_y = lax.axis_index("y")
        my_z = lax.axis_index("z")
        peer = (1 - my_x, my_y, my_z)

        barrier_sem = pltpu.get_barrier_semaphore()
        pl.semaphore_signal(barrier_sem, inc=1, device_id=peer,
                            device_id_type=pl.DeviceIdType.MESH)
        pl.semaphore_wait(barrier_sem, 1)

        my_base = my_x * V_LOC
        peer_base = (1 - my_x) * V_LOC

        m_ref[...] = jnp.full((T, 1), -1e30, jnp.float32)
        s_ref[...] = jnp.zeros((T, 1), jnp.float32)

        def w_copy(k):
            return pltpu.make_async_copy(
                w_hbm.at[:, pl.ds(k * CHUNK, CHUNK)],
                w_buf.at[k % 2],
                w_sems.at[k % 2],
            )

        def store_copy(k):
            return pltpu.make_async_copy(
                l_buf.at[k % 2],
                out_hbm.at[:, pl.ds(my_base + k * CHUNK, CHUNK)],
                st_sems.at[k % 2],
            )

        def chunk_rdma(k):
            return pltpu.make_async_remote_copy(
                src_ref=l_buf.at[k % 2],
                dst_ref=out_hbm.at[:, pl.ds(my_base + k * CHUNK, CHUNK)],
                send_sem=send_sems.at[k],
                recv_sem=recv_sems.at[k],
                device_id=peer,
                device_id_type=pl.DeviceIdType.MESH,
            )

        w_copy(0).start()

        def mm_body(k, carry):
            slot = k % 2

            @pl.when(k + 1 < N_CHUNK)
            def _():
                w_copy(k + 1).start()

            w_copy(k).wait()

            @pl.when(k >= 2)
            def _():
                chunk_rdma(k - 2).wait_send()
                store_copy(k - 2).wait()

            logits = jnp.dot(x_ref[...], w_buf[slot],
                             preferred_element_type=jnp.float32)
            m_old = m_ref[...]
            m_new = jnp.maximum(m_old,
                                jnp.max(logits, axis=1, keepdims=True))
            e_sum = jnp.sum(jnp.exp(logits - m_new), axis=1, keepdims=True)
            s_ref[...] = s_ref[...] * jnp.exp(m_old - m_new) + e_sum
            m_ref[...] = m_new
            l_buf[slot] = logits

            store_copy(k).start()
            chunk_rdma(k).start()
            return carry

        lax.fori_loop(0, N_CHUNK, mm_body, 0)

        for k in (N_CHUNK - 2, N_CHUNK - 1):
            chunk_rdma(k).wait_send()
            store_copy(k).wait()

        ms_send[:, 0:1] = m_ref[...]
        ms_send[:, 1:2] = s_ref[...]
        ms_rdma = pltpu.make_async_remote_copy(
            src_ref=ms_send,
            dst_ref=ms_recv,
            send_sem=ms_send_sem,
            recv_sem=ms_recv_sem,
            device_id=peer,
            device_id_type=pl.DeviceIdType.MESH,
        )
        ms_rdma.start()
        ms_rdma.wait()

        m_loc = m_ref[...]
        s_loc = s_ref[...]
        m_rem = ms_recv[:, 0:1]
        s_rem = ms_recv[:, 1:2]
        m_g = jnp.maximum(m_loc, m_rem)
        s_g = s_loc * jnp.exp(m_loc - m_g) + s_rem * jnp.exp(m_rem - m_g)
        inv_s = 1.0 / s_g

        n_buf = l_buf
        N2 = 2 * N_CHUNK

        def nm_col(j):
            return jnp.where(j < N_CHUNK,
                             my_base + j * CHUNK,
                             peer_base + (j - N_CHUNK) * CHUNK)

        def nm_body(j, carry):
            slot = j % 2

            @pl.when(j >= N_CHUNK)
            def _():
                chunk_rdma(j - N_CHUNK).wait_recv()

            @pl.when(j >= 2)
            def _():
                pltpu.make_async_copy(
                    n_buf.at[slot],
                    out_hbm.at[:, pl.ds(nm_col(j - 2), CHUNK)],
                    wr_sems.at[slot],
                ).wait()

            rd = pltpu.make_async_copy(
                out_hbm.at[:, pl.ds(nm_col(j), CHUNK)],
                n_buf.at[slot],
                rd_sems.at[slot],
            )
            rd.start()
            rd.wait()
            n_buf[slot] = jnp.exp(n_buf[slot] - m_g) * inv_s
            pltpu.make_async_copy(
                n_buf.at[slot],
                out_hbm.at[:, pl.ds(nm_col(j), CHUNK)],
                wr_sems.at[slot],
            ).start()
            return carry

        lax.fori_loop(0, N2, nm_body, 0)

        for j in (N2 - 2, N2 - 1):
            pltpu.make_async_copy(
                n_buf.at[j % 2],
                out_hbm.at[:, pl.ds(nm_col(j), CHUNK)],
                wr_sems.at[j % 2],
            ).wait()

    return pl.pallas_call(
        body,
        out_shape=jax.ShapeDtypeStruct((T, V_GLOB), jnp.float32),
        in_specs=[
            pl.BlockSpec(memory_space=pltpu.VMEM),
            pl.BlockSpec(memory_space=pltpu.MemorySpace.HBM),
        ],
        out_specs=pl.BlockSpec(memory_space=pltpu.MemorySpace.HBM),
        scratch_shapes=[
            pltpu.VMEM((2, D, CHUNK), jnp.float32),
            pltpu.VMEM((2, T, CHUNK), jnp.float32),
            pltpu.VMEM((T, 1), jnp.float32),
            pltpu.VMEM((T, 1), jnp.float32),
            pltpu.VMEM((T, 8), jnp.float32),
            pltpu.VMEM((T, 8), jnp.float32),
            pltpu.SemaphoreType.DMA((2,)),
            pltpu.SemaphoreType.DMA((2,)),
            pltpu.SemaphoreType.DMA((N_CHUNK,)),
            pltpu.SemaphoreType.DMA((N_CHUNK,)),
            pltpu.SemaphoreType.DMA,
            pltpu.SemaphoreType.DMA,
            pltpu.SemaphoreType.DMA((2,)),
            pltpu.SemaphoreType.DMA((2,)),
        ],
        compiler_params=pltpu.CompilerParams(collective_id=0),
    )(x, W)
